# baseline (speedup 1.0000x reference)
import functools

import jax
import jax.numpy as jnp
from jax import lax
from jax.experimental import pallas as pl
from jax.experimental.pallas import tpu as pltpu
from jax.experimental.pallas import tpu_sc as plsc

B, H, W, C = 32, 56, 56, 384
HW = H * W
N = B * HW
EPS = 1e-3


_ROWS_PER_WORKER = 8
_NUM_WORKERS = B // _ROWS_PER_WORKER


@functools.cache
def _make_sc_gather():
    @functools.partial(
        pl.kernel,
        out_type=jax.ShapeDtypeStruct((2 * B, C), jnp.float32),
        mesh=plsc.VectorSubcoreMesh(core_axis_name="c", subcore_axis_name="s"),
        scratch_types=[
            pltpu.VMEM((_ROWS_PER_WORKER,), jnp.int32),
            pltpu.VMEM((_ROWS_PER_WORKER, C), jnp.float32),
            pltpu.SemaphoreType.DMA,
        ],
    )
    def _sc_gather(labels_hbm, gamma_hbm, beta_hbm, gb_out,
                   idx_v, rows, sem):
        wid = lax.axis_index("s") * 2 + lax.axis_index("c")

        @pl.when(wid < _NUM_WORKERS)
        def _():
            base = wid * _ROWS_PER_WORKER
            pltpu.sync_copy(labels_hbm.at[pl.ds(base, _ROWS_PER_WORKER)], idx_v)
            pltpu.async_copy(gamma_hbm.at[idx_v], rows, sem).wait()
            pltpu.sync_copy(rows, gb_out.at[pl.ds(base, _ROWS_PER_WORKER)])

        @pl.when((wid >= _NUM_WORKERS) & (wid < 2 * _NUM_WORKERS))
        def _():
            base = (wid - _NUM_WORKERS) * _ROWS_PER_WORKER
            pltpu.sync_copy(labels_hbm.at[pl.ds(base, _ROWS_PER_WORKER)], idx_v)
            pltpu.async_copy(beta_hbm.at[idx_v], rows, sem).wait()
            pltpu.sync_copy(rows, gb_out.at[pl.ds(B + base, _ROWS_PER_WORKER)])

    return _sc_gather


K_RES = 18


def _fused_body(bng_ref, bnb_ref, g_ref, b_ref, x_ref, o_ref, acc_ref, res_ref):
    p = pl.program_id(0)
    i = pl.program_id(1)

    @pl.when(p == 0)
    def _():
        x = x_ref[0]
        nchunk = 4
        rows = HW // nchunk
        ones8 = jnp.ones((8, rows), jnp.float32)
        s8 = jnp.zeros((8, C), jnp.float32)
        ss8 = jnp.zeros((8, C), jnp.float32)
        for k in range(nchunk):
            xk = x[k * rows:(k + 1) * rows]
            s8 = s8 + jax.lax.dot(ones8, xk,
                                  precision=lax.Precision.HIGHEST,
                                  preferred_element_type=jnp.float32)
            ss8 = ss8 + jax.lax.dot(ones8, xk * xk,
                                    precision=lax.Precision.HIGHEST,
                                    preferred_element_type=jnp.float32)
        s = s8[0:1]
        ss = ss8[0:1]

        @pl.when(i == 0)
        def _():
            acc_ref[0:1, :] = s
            acc_ref[1:2, :] = ss

        @pl.when(i > 0)
        def _():
            acc_ref[0:1, :] += s
            acc_ref[1:2, :] += ss

        @pl.when(i >= B - K_RES)
        def _():
            slot = jnp.maximum(i - (B - K_RES), 0)
            res_ref[pl.ds(slot, 1)] = x_ref[...].astype(jnp.bfloat16)

    @pl.when(p == 1)
    def _():
        inv_n = jnp.float32(1.0 / N)
        mu = acc_ref[0:1, :] * inv_n
        var = acc_ref[1:2, :] * inv_n - mu * mu
        rstd = lax.rsqrt(var + jnp.float32(EPS))
        a = rstd * bng_ref[0:1, :]
        g = g_ref[0]
        scale = a * g
        shift = (bnb_ref[0:1, :] - mu * a) * g + b_ref[0]

        @pl.when(i < K_RES)
        def _():
            slot = jnp.minimum(i, K_RES - 1)
            xr = res_ref[pl.ds(slot, 1)][0].astype(jnp.float32)
            o_ref[0] = xr * scale + shift

        @pl.when(i >= K_RES)
        def _():
            o_ref[0] = x_ref[0] * scale + shift


def _out_idx(p, i):
    ph1 = jnp.where(i < K_RES, B - K_RES + i, i - K_RES)
    return jnp.where(p == 0, B - K_RES, ph1)


def _x_idx(p, i):
    ph1 = jnp.where(i < K_RES, 0, i - K_RES)
    return jnp.where(p == 0, i, ph1)


def kernel(inputs, labels, gamma, beta, bn_gamma, bn_beta):
    x3 = inputs.reshape(B, HW, C)
    labels_i = labels.astype(jnp.int32)

    gb_rows = _make_sc_gather()(labels_i, gamma, beta)

    g3 = gb_rows[:B].reshape(B, 1, C)
    b3 = gb_rows[B:].reshape(B, 1, C)
    out = pl.pallas_call(
        _fused_body,
        grid=(2, B),
        in_specs=[
            pl.BlockSpec((1, C), lambda p, i: (0, 0)),
            pl.BlockSpec((1, C), lambda p, i: (0, 0)),
            pl.BlockSpec((1, 1, C), lambda p, i: (_out_idx(p, i), 0, 0)),
            pl.BlockSpec((1, 1, C), lambda p, i: (_out_idx(p, i), 0, 0)),
            pl.BlockSpec((1, HW, C), lambda p, i: (_x_idx(p, i), 0, 0)),
        ],
        out_specs=pl.BlockSpec((1, HW, C), lambda p, i: (_out_idx(p, i), 0, 0)),
        out_shape=jax.ShapeDtypeStruct((B, HW, C), jnp.float32),
        scratch_shapes=[
            pltpu.VMEM((2, C), jnp.float32),
            pltpu.VMEM((K_RES, HW, C), jnp.bfloat16),
        ],
        compiler_params=pltpu.CompilerParams(
            vmem_limit_bytes=64 * 1024 * 1024,
        ),
    )(bn_gamma.reshape(1, C), bn_beta.reshape(1, C), g3, b3, x3)

    return out.reshape(B, H, W, C)


# speedup vs baseline: 1.5884x; 1.5884x over previous
import functools

import jax
import jax.numpy as jnp
from jax import lax
from jax.experimental import pallas as pl
from jax.experimental.pallas import tpu as pltpu
from jax.experimental.pallas import tpu_sc as plsc

B, H, W, C = 32, 56, 56, 384
HW = H * W
N = B * HW
EPS = 1e-3


_ROWS_PER_WORKER = 8
_NUM_WORKERS = B // _ROWS_PER_WORKER


@functools.cache
def _make_sc_gather():
    @functools.partial(
        pl.kernel,
        out_type=jax.ShapeDtypeStruct((2 * B, C), jnp.float32),
        mesh=plsc.VectorSubcoreMesh(core_axis_name="c", subcore_axis_name="s"),
        scratch_types=[
            pltpu.VMEM((_ROWS_PER_WORKER,), jnp.int32),
            pltpu.VMEM((_ROWS_PER_WORKER, C), jnp.float32),
            pltpu.SemaphoreType.DMA,
        ],
    )
    def _sc_gather(labels_hbm, gamma_hbm, beta_hbm, gb_out,
                   idx_v, rows, sem):
        wid = lax.axis_index("s") * 2 + lax.axis_index("c")

        @pl.when(wid < _NUM_WORKERS)
        def _():
            base = wid * _ROWS_PER_WORKER
            pltpu.sync_copy(labels_hbm.at[pl.ds(base, _ROWS_PER_WORKER)], idx_v)
            pltpu.async_copy(gamma_hbm.at[idx_v], rows, sem).wait()
            pltpu.sync_copy(rows, gb_out.at[pl.ds(base, _ROWS_PER_WORKER)])

        @pl.when((wid >= _NUM_WORKERS) & (wid < 2 * _NUM_WORKERS))
        def _():
            base = (wid - _NUM_WORKERS) * _ROWS_PER_WORKER
            pltpu.sync_copy(labels_hbm.at[pl.ds(base, _ROWS_PER_WORKER)], idx_v)
            pltpu.async_copy(beta_hbm.at[idx_v], rows, sem).wait()
            pltpu.sync_copy(rows, gb_out.at[pl.ds(B + base, _ROWS_PER_WORKER)])

    return _sc_gather


K_RES = 18


def _fused_body(bng_ref, bnb_ref, g_ref, b_ref, x_ref, o_ref, acc_ref, res_ref):
    p = pl.program_id(0)
    i = pl.program_id(1)

    @pl.when(p == 0)
    def _():
        x = x_ref[0]
        s = jnp.sum(x, axis=0, keepdims=True)
        ss = jnp.sum(x * x, axis=0, keepdims=True)

        @pl.when(i == 0)
        def _():
            acc_ref[0:1, :] = s
            acc_ref[1:2, :] = ss

        @pl.when(i > 0)
        def _():
            acc_ref[0:1, :] += s
            acc_ref[1:2, :] += ss

        @pl.when(i >= B - K_RES)
        def _():
            slot = jnp.maximum(i - (B - K_RES), 0)
            res_ref[pl.ds(slot, 1)] = x_ref[...].astype(jnp.bfloat16)

    @pl.when(p == 1)
    def _():
        inv_n = jnp.float32(1.0 / N)
        mu = acc_ref[0:1, :] * inv_n
        var = acc_ref[1:2, :] * inv_n - mu * mu
        rstd = lax.rsqrt(var + jnp.float32(EPS))
        a = rstd * bng_ref[0:1, :]
        g = g_ref[0]
        scale = a * g
        shift = (bnb_ref[0:1, :] - mu * a) * g + b_ref[0]

        @pl.when(i < K_RES)
        def _():
            slot = jnp.minimum(i, K_RES - 1)
            xr = res_ref[pl.ds(slot, 1)][0].astype(jnp.float32)
            o_ref[0] = xr * scale + shift

        @pl.when(i >= K_RES)
        def _():
            o_ref[0] = x_ref[0] * scale + shift


def _out_idx(p, i):
    ph1 = jnp.where(i < K_RES, B - K_RES + i, i - K_RES)
    return jnp.where(p == 0, B - K_RES, ph1)


def _x_idx(p, i):
    ph1 = jnp.where(i < K_RES, 0, i - K_RES)
    return jnp.where(p == 0, i, ph1)


def kernel(inputs, labels, gamma, beta, bn_gamma, bn_beta):
    x3 = inputs.reshape(B, HW, C)
    labels_i = labels.astype(jnp.int32)

    gb_rows = _make_sc_gather()(labels_i, gamma, beta)

    g3 = gb_rows[:B].reshape(B, 1, C)
    b3 = gb_rows[B:].reshape(B, 1, C)
    out = pl.pallas_call(
        _fused_body,
        grid=(2, B),
        in_specs=[
            pl.BlockSpec((1, C), lambda p, i: (0, 0)),
            pl.BlockSpec((1, C), lambda p, i: (0, 0)),
            pl.BlockSpec((1, 1, C), lambda p, i: (_out_idx(p, i), 0, 0)),
            pl.BlockSpec((1, 1, C), lambda p, i: (_out_idx(p, i), 0, 0)),
            pl.BlockSpec((1, HW, C), lambda p, i: (_x_idx(p, i), 0, 0)),
        ],
        out_specs=pl.BlockSpec((1, HW, C), lambda p, i: (_out_idx(p, i), 0, 0)),
        out_shape=jax.ShapeDtypeStruct((B, HW, C), jnp.float32),
        scratch_shapes=[
            pltpu.VMEM((2, C), jnp.float32),
            pltpu.VMEM((K_RES, HW, C), jnp.bfloat16),
        ],
        compiler_params=pltpu.CompilerParams(
            vmem_limit_bytes=64 * 1024 * 1024,
        ),
    )(bn_gamma.reshape(1, C), bn_beta.reshape(1, C), g3, b3, x3)

    return out.reshape(B, H, W, C)
